# baseline (speedup 1.0000x reference)
import functools

import jax
import jax.numpy as jnp
from jax import lax
from jax.experimental import pallas as pl
from jax.experimental.pallas import tpu as pltpu
from jax.experimental.pallas import tpu_sc as plsc

B, S, D = 4, 8192, 2048
L = 16


def _pooler(h_rows, tokens):
    mesh = plsc.VectorSubcoreMesh(core_axis_name="c", subcore_axis_name="s",
                                  num_cores=1)

    @functools.partial(
        pl.kernel,
        out_type=jax.ShapeDtypeStruct((B, D), jnp.float32),
        mesh=mesh,
        compiler_params=pltpu.CompilerParams(needs_layout_passes=False,
                                             skip_device_barrier=True),
        scratch_types=[
            pltpu.VMEM((S,), jnp.int32),
        ],
    )
    def k(h_hbm, tok_hbm, out_hbm, row_v):
        sid = lax.axis_index("s")

        @pl.when(sid < B)
        def _():
            b = sid
            pltpu.sync_copy(tok_hbm.at[b], row_v)

            U = 16

            def body(i, acc):
                base = i * (L * U)
                for u in range(U):
                    x = plsc.bitcast(row_v[pl.ds(base + u * L, L)], jnp.uint32)
                    acc = acc + jnp.minimum(x, 1)
                return acc

            lane_cnt = lax.fori_loop(0, S // (L * U), body,
                                     jnp.zeros((L,), jnp.uint32))
            cnt = jnp.sum(lane_cnt).astype(jnp.int32)
            idx = cnt - 1
            idx = jnp.where(idx < 0, idx + S, idx)
            pltpu.sync_copy(h_hbm.at[b * S + idx], out_hbm.at[b])

    return k(h_rows, tokens)


def kernel(h, inputs):
    return _pooler(h.reshape(B * S, D), inputs)


# speedup vs baseline: 1.0096x; 1.0096x over previous
import functools

import jax
import jax.numpy as jnp
from jax import lax
from jax.experimental import pallas as pl
from jax.experimental.pallas import tpu as pltpu
from jax.experimental.pallas import tpu_sc as plsc

B, S, D = 4, 8192, 2048
L = 16


def _pooler(h_rows, tokens):
    mesh = plsc.VectorSubcoreMesh(core_axis_name="c", subcore_axis_name="s",
                                  num_cores=1, num_subcores=B)

    @functools.partial(
        pl.kernel,
        out_type=jax.ShapeDtypeStruct((B, D), jnp.float32),
        mesh=mesh,
        compiler_params=pltpu.CompilerParams(needs_layout_passes=False,
                                             skip_device_barrier=True),
        scratch_types=[
            pltpu.VMEM((S,), jnp.int32),
        ],
    )
    def k(h_hbm, tok_hbm, out_hbm, row_v):
        sid = lax.axis_index("s")

        @pl.when(sid < B)
        def _():
            b = sid
            pltpu.sync_copy(tok_hbm.at[b], row_v)

            U = 8

            def body(i, acc):
                base = i * (L * U)
                for u in range(U):
                    x = row_v[pl.ds(base + u * L, L)]
                    acc = acc + (x != 0).astype(jnp.int32)
                return acc

            lane_cnt = lax.fori_loop(0, S // (L * U), body,
                                     jnp.zeros((L,), jnp.int32))
            cnt = jnp.sum(lane_cnt)
            idx = cnt - 1
            idx = jnp.where(idx < 0, idx + S, idx)
            pltpu.sync_copy(h_hbm.at[b * S + idx], out_hbm.at[b])

    return k(h_rows, tokens)


def kernel(h, inputs):
    return _pooler(h.reshape(B * S, D), inputs)


# speedup vs baseline: 1.0180x; 1.0083x over previous
import functools

import jax
import jax.numpy as jnp
from jax import lax
from jax.experimental import pallas as pl
from jax.experimental.pallas import tpu as pltpu
from jax.experimental.pallas import tpu_sc as plsc

B, S, D = 4, 8192, 2048
L = 16
W = 4
Q = S // W


def _pooler(h_rows, tokens):
    mesh = plsc.VectorSubcoreMesh(core_axis_name="c", subcore_axis_name="s",
                                  num_cores=1)

    @functools.partial(
        pl.kernel,
        out_type=jax.ShapeDtypeStruct((B, D), jnp.float32),
        mesh=mesh,
        compiler_params=pltpu.CompilerParams(needs_layout_passes=False,
                                             skip_device_barrier=True),
        scratch_types=[
            pltpu.VMEM((Q,), jnp.int32),
            pltpu.VMEM((L,), jnp.int32),
            pltpu.VMEM((W * L,), jnp.int32),
            pltpu.VMEM_SHARED((B * W * L,), jnp.int32),
        ],
    )
    def k(h_hbm, tok_hbm, out_hbm, row_v, part_v, sums_v, shared):
        t = lax.axis_index("s")
        b1 = t // W
        q = t % W
        pltpu.sync_copy(tok_hbm.at[b1, pl.ds(q * Q, Q)], row_v)

        U = 8

        def body(i, acc):
            base = i * (L * U)
            for u in range(U):
                x = row_v[pl.ds(base + u * L, L)]
                acc = acc + (x != 0).astype(jnp.int32)
            return acc

        part_v[...] = lax.fori_loop(0, Q // (L * U), body,
                                    jnp.zeros((L,), jnp.int32))
        pltpu.sync_copy(part_v, shared.at[pl.ds(t * L, L)])
        plsc.subcore_barrier()

        @pl.when(t < B)
        def _():
            b = t
            pltpu.sync_copy(shared.at[pl.ds(b * W * L, W * L)], sums_v)
            lane_cnt = jnp.zeros((L,), jnp.int32)
            for j in range(W):
                lane_cnt = lane_cnt + sums_v[pl.ds(j * L, L)]
            cnt = jnp.sum(lane_cnt)
            idx = cnt - 1
            idx = jnp.where(idx < 0, idx + S, idx)
            pltpu.sync_copy(h_hbm.at[b * S + idx], out_hbm.at[b])

    return k(h_rows, tokens)


def kernel(h, inputs):
    return _pooler(h.reshape(B * S, D), inputs)


# speedup vs baseline: 1.0188x; 1.0008x over previous
import functools

import jax
import jax.numpy as jnp
from jax import lax
from jax.experimental import pallas as pl
from jax.experimental.pallas import tpu as pltpu
from jax.experimental.pallas import tpu_sc as plsc

B, S, D = 4, 8192, 2048
L = 16


def _pooler(h_rows, tokens):
    mesh = plsc.VectorSubcoreMesh(core_axis_name="c", subcore_axis_name="s",
                                  num_cores=1, num_subcores=B)

    @functools.partial(
        pl.kernel,
        out_type=jax.ShapeDtypeStruct((B, D), jnp.float32),
        mesh=mesh,
        compiler_params=pltpu.CompilerParams(needs_layout_passes=False,
                                             skip_device_barrier=True),
        scratch_types=[
            pltpu.VMEM((S,), jnp.int32),
        ],
    )
    def k(h_hbm, tok_hbm, out_hbm, row_v):
        sid = lax.axis_index("s")

        @pl.when(sid < B)
        def _():
            b = sid
            pltpu.sync_copy(tok_hbm.at[b], row_v)

            U = 8

            def body(i, acc):
                base = i * (L * U)
                for u in range(U):
                    x = row_v[pl.ds(base + u * L, L)]
                    acc = acc + (x != 0).astype(jnp.int32)
                return acc

            lane_cnt = lax.fori_loop(0, S // (L * U), body,
                                     jnp.zeros((L,), jnp.int32))
            cnt = jnp.sum(lane_cnt)
            idx = cnt - 1
            idx = jnp.where(idx < 0, idx + S, idx)
            pltpu.sync_copy(h_hbm.at[b * S + idx], out_hbm.at[b])

    return k(h_rows, tokens)


def kernel(h, inputs):
    return _pooler(h.reshape(B * S, D), inputs)
